# baseline (speedup 1.0000x reference)
import functools

import jax
import jax.numpy as jnp
import numpy as np
from jax import lax
from jax.experimental import pallas as pl
from jax.experimental.pallas import tpu as pltpu
from jax.experimental.pallas import tpu_sc as plsc

B = 16384
F = 26
D = 16
BF = B * F

NC = 2
NS = 16
NW = NC * NS
HALVES = 2
BH = B // HALVES
BFH = BH * F
PER_W = BFH // NW
CH = 1664
NCHUNK = PER_W // CH


def _sc_gather(ids, fmw, fmv):
    mesh = plsc.VectorSubcoreMesh(core_axis_name="c", subcore_axis_name="s")

    @functools.partial(
        pl.kernel,
        mesh=mesh,
        compiler_params=pltpu.CompilerParams(use_tc_tiling_on_sc=False),
        out_type=(
            jax.ShapeDtypeStruct((BFH, D), jnp.float32),
            jax.ShapeDtypeStruct((BFH,), jnp.float32),
        ),
        scratch_types=[
            pltpu.VMEM((PER_W,), jnp.int32),
            pltpu.VMEM((CH, D), jnp.float32),
            pltpu.VMEM((CH, D), jnp.float32),
            pltpu.VMEM((PER_W,), jnp.float32),
            pltpu.SemaphoreType.DMA,
            pltpu.SemaphoreType.DMA,
            pltpu.SemaphoreType.DMA,
            pltpu.SemaphoreType.DMA,
            pltpu.SemaphoreType.DMA,
        ],
    )
    def gk(ids_hbm, fmw_hbm, fmv_hbm, emb_hbm, w_hbm,
           idx_v, rows_v0, rows_v1, w_v, semg0, semg1, semo0, semo1, sem_w):
        wid = lax.axis_index("s") * NC + lax.axis_index("c")
        base = wid * PER_W
        rows = [rows_v0, rows_v1]
        semg = [semg0, semg1]
        semo = [semo0, semo1]
        pltpu.sync_copy(ids_hbm.at[pl.ds(base, PER_W)], idx_v)
        wcopy = pltpu.async_copy(fmw_hbm.at[idx_v], w_v, sem_w)

        def gstart(c):
            return pltpu.async_copy(
                fmv_hbm.at[idx_v.at[pl.ds(c * CH, CH)]], rows[c % 2],
                semg[c % 2])

        gcopies = {0: gstart(0)}
        ocopies = {}
        for c in range(NCHUNK):
            b = c % 2
            if c + 1 < NCHUNK:
                gcopies[c + 1] = gstart(c + 1)
            gcopies[c].wait()
            if c >= 2:
                ocopies[c - 2].wait()
            ocopies[c] = pltpu.async_copy(
                rows[b], emb_hbm.at[pl.ds(base + c * CH, CH)], semo[b])
        ocopies[NCHUNK - 2].wait()
        ocopies[NCHUNK - 1].wait()
        wcopy.wait()
        pltpu.sync_copy(w_v, w_hbm.at[pl.ds(base, PER_W)])

    return gk(ids, fmw, fmv)


BLK = 1024
GRID = BH // BLK


def _tc_mlp(emb, vals, w, S, T, W0, b0, W1, b1, W2, b2, Wo, bfin):
    def mk(emb_r, vals_r, w_r, S_r, T_r, W0r, b0r, W1r, b1r, W2r, b2r, Wor,
           bfr, out_r):
        vals_b = vals_r[...]
        y_w = jnp.sum(w_r[...] * vals_b, axis=1, keepdims=True)
        vr = jnp.dot(vals_b, S_r[...], preferred_element_type=jnp.float32,
                     precision=lax.Precision.HIGHEST)
        x = emb_r[...] * vr
        sv = jnp.dot(x, T_r[...], preferred_element_type=jnp.float32,
                     precision=lax.Precision.HIGHEST)
        y_v = 0.5 * (jnp.sum(sv * sv, axis=1, keepdims=True)
                     - jnp.sum(x * x, axis=1, keepdims=True))
        xb = x.astype(jnp.bfloat16)
        h = jnp.maximum(jnp.dot(xb, W0r[...],
                                preferred_element_type=jnp.float32) + b0r[...], 0.0)
        h = jnp.maximum(jnp.dot(h.astype(jnp.bfloat16), W1r[...],
                                preferred_element_type=jnp.float32) + b1r[...], 0.0)
        h = jnp.maximum(jnp.dot(h.astype(jnp.bfloat16), W2r[...],
                                preferred_element_type=jnp.float32) + b2r[...], 0.0)
        y_d = jnp.dot(h, Wor[...], preferred_element_type=jnp.float32)
        y = y_w + y_v + y_d + bfr[...]
        out_r[...] = jax.nn.sigmoid(y)

    full = lambda a: pl.BlockSpec(a.shape, lambda i: (0,) * a.ndim)
    return pl.pallas_call(
        mk,
        grid=(GRID,),
        in_specs=[
            pl.BlockSpec((BLK, F * D), lambda i: (i, 0)),
            pl.BlockSpec((BLK, F), lambda i: (i, 0)),
            pl.BlockSpec((BLK, F), lambda i: (i, 0)),
            full(S), full(T), full(W0), full(b0), full(W1), full(b1),
            full(W2), full(b2), full(Wo), full(bfin),
        ],
        out_specs=pl.BlockSpec((BLK, 1), lambda i: (i, 0)),
        out_shape=jax.ShapeDtypeStruct((BH, 1), jnp.float32),
    )(emb, vals, w, S, T, W0, b0, W1, b1, W2, b2, Wo, bfin)


def kernel(feat_ids, feat_vals, FM_B, FM_W, FM_V, params):
    a = [params[f"gamma{i}"] * lax.rsqrt(params[f"var{i}"] + 1e-3)
         for i in range(3)]
    c = [params[f"beta{i}"] - params[f"mean{i}"] * a[i] for i in range(3)]
    W0, b0 = params["W0"], params["b0"]
    W1 = a[0][:, None] * params["W1"]
    b1 = c[0] @ params["W1"] + params["b1"]
    W2 = a[1][:, None] * params["W2"]
    b2 = c[1] @ params["W2"] + params["b2"]
    Wo = a[2][:, None] * params["W_out"]
    bfin = c[2] @ params["W_out"] + params["b_out"] + FM_B

    S = jnp.asarray(np.repeat(np.eye(F, dtype=np.float32), D, axis=1))
    T = jnp.asarray(np.tile(np.eye(D, dtype=np.float32), (F, 1)))

    ids = feat_ids.reshape(HALVES, BFH).astype(jnp.int32)
    gathered = [_sc_gather(ids[k], FM_W, FM_V) for k in range(HALVES)]
    preds = []
    for k in range(HALVES):
        emb_flat, w_flat = gathered[k]
        preds.append(_tc_mlp(
            emb_flat.reshape(BH, F * D),
            lax.dynamic_slice_in_dim(feat_vals, k * BH, BH),
            w_flat.reshape(BH, F), S, T,
            W0.astype(jnp.bfloat16), b0.reshape(1, -1),
            W1.astype(jnp.bfloat16), b1.reshape(1, -1),
            W2.astype(jnp.bfloat16), b2.reshape(1, -1), Wo,
            bfin.reshape(1, 1),
        ))
    return jnp.concatenate(preds, axis=0).reshape(-1)


# speedup vs baseline: 1.0124x; 1.0124x over previous
import functools

import jax
import jax.numpy as jnp
import numpy as np
from jax import lax
from jax.experimental import pallas as pl
from jax.experimental.pallas import tpu as pltpu
from jax.experimental.pallas import tpu_sc as plsc

B = 16384
F = 26
D = 16
BF = B * F

NC = 2
NS = 16
NW = NC * NS
PER_W = BF // NW
CH = 1664
NCHUNK = PER_W // CH


def _sc_gather(ids, fmw, fmv):
    mesh = plsc.VectorSubcoreMesh(core_axis_name="c", subcore_axis_name="s")

    @functools.partial(
        pl.kernel,
        mesh=mesh,
        compiler_params=pltpu.CompilerParams(use_tc_tiling_on_sc=False),
        out_type=(
            jax.ShapeDtypeStruct((BF, D), jnp.float32),
            jax.ShapeDtypeStruct((BF,), jnp.float32),
        ),
        scratch_types=[
            pltpu.VMEM((PER_W,), jnp.int32),
            pltpu.VMEM((CH, D), jnp.float32),
            pltpu.VMEM((CH, D), jnp.float32),
            pltpu.VMEM((PER_W,), jnp.float32),
            pltpu.SemaphoreType.DMA,
            pltpu.SemaphoreType.DMA,
            pltpu.SemaphoreType.DMA,
            pltpu.SemaphoreType.DMA,
            pltpu.SemaphoreType.DMA,
        ],
    )
    def gk(ids_hbm, fmw_hbm, fmv_hbm, emb_hbm, w_hbm,
           idx_v, rows_v0, rows_v1, w_v, semg0, semg1, semo0, semo1, sem_w):
        wid = lax.axis_index("s") * NC + lax.axis_index("c")
        base = wid * PER_W
        rows = [rows_v0, rows_v1]
        semg = [semg0, semg1]
        semo = [semo0, semo1]
        pltpu.sync_copy(ids_hbm.at[pl.ds(base, PER_W)], idx_v)
        wcopy = pltpu.async_copy(fmw_hbm.at[idx_v], w_v, sem_w)

        def gstart(c):
            return pltpu.async_copy(
                fmv_hbm.at[idx_v.at[pl.ds(c * CH, CH)]], rows[c % 2],
                semg[c % 2])

        gcopies = {0: gstart(0)}
        ocopies = {}
        for c in range(NCHUNK):
            b = c % 2
            if c + 1 < NCHUNK:
                gcopies[c + 1] = gstart(c + 1)
            gcopies[c].wait()
            if c >= 2:
                ocopies[c - 2].wait()
            ocopies[c] = pltpu.async_copy(
                rows[b], emb_hbm.at[pl.ds(base + c * CH, CH)], semo[b])
        ocopies[NCHUNK - 2].wait()
        ocopies[NCHUNK - 1].wait()
        wcopy.wait()
        pltpu.sync_copy(w_v, w_hbm.at[pl.ds(base, PER_W)])

    return gk(ids, fmw, fmv)


BLK = 1024
GRID = B // BLK


def _tc_mlp(emb, vals, w, S, T, W0, b0, W1, b1, W2, b2, Wo, bfin):
    def mk(emb_r, vals_r, w_r, S_r, T_r, W0r, b0r, W1r, b1r, W2r, b2r, Wor,
           bfr, out_r):
        vals_b = vals_r[...]
        y_w = jnp.sum(w_r[...] * vals_b, axis=1, keepdims=True)
        vr = jnp.dot(vals_b, S_r[...], preferred_element_type=jnp.float32,
                     precision=lax.Precision.HIGHEST)
        x = emb_r[...] * vr
        sv = jnp.dot(x, T_r[...], preferred_element_type=jnp.float32,
                     precision=lax.Precision.HIGHEST)
        y_v = 0.5 * (jnp.sum(sv * sv, axis=1, keepdims=True)
                     - jnp.sum(x * x, axis=1, keepdims=True))
        xb = x.astype(jnp.bfloat16)
        h = jnp.maximum(jnp.dot(xb, W0r[...],
                                preferred_element_type=jnp.float32) + b0r[...], 0.0)
        h = jnp.maximum(jnp.dot(h.astype(jnp.bfloat16), W1r[...],
                                preferred_element_type=jnp.float32) + b1r[...], 0.0)
        h = jnp.maximum(jnp.dot(h.astype(jnp.bfloat16), W2r[...],
                                preferred_element_type=jnp.float32) + b2r[...], 0.0)
        y_d = jnp.dot(h, Wor[...], preferred_element_type=jnp.float32)
        y = y_w + y_v + y_d + bfr[...]
        out_r[...] = jax.nn.sigmoid(y)

    full = lambda a: pl.BlockSpec(a.shape, lambda i: (0,) * a.ndim)
    return pl.pallas_call(
        mk,
        grid=(GRID,),
        in_specs=[
            pl.BlockSpec((BLK, F * D), lambda i: (i, 0)),
            pl.BlockSpec((BLK, F), lambda i: (i, 0)),
            pl.BlockSpec((BLK, F), lambda i: (i, 0)),
            full(S), full(T), full(W0), full(b0), full(W1), full(b1),
            full(W2), full(b2), full(Wo), full(bfin),
        ],
        out_specs=pl.BlockSpec((BLK, 1), lambda i: (i, 0)),
        out_shape=jax.ShapeDtypeStruct((B, 1), jnp.float32),
    )(emb, vals, w, S, T, W0, b0, W1, b1, W2, b2, Wo, bfin)


def kernel(feat_ids, feat_vals, FM_B, FM_W, FM_V, params):
    ids = feat_ids.reshape(-1).astype(jnp.int32)
    emb_flat, w_flat = _sc_gather(ids, FM_W, FM_V)
    emb = emb_flat.reshape(B, F * D)
    w = w_flat.reshape(B, F)

    a = [params[f"gamma{i}"] * lax.rsqrt(params[f"var{i}"] + 1e-3)
         for i in range(3)]
    c = [params[f"beta{i}"] - params[f"mean{i}"] * a[i] for i in range(3)]
    W0, b0 = params["W0"], params["b0"]
    W1 = a[0][:, None] * params["W1"]
    b1 = c[0] @ params["W1"] + params["b1"]
    W2 = a[1][:, None] * params["W2"]
    b2 = c[1] @ params["W2"] + params["b2"]
    Wo = a[2][:, None] * params["W_out"]
    bfin = c[2] @ params["W_out"] + params["b_out"] + FM_B

    S = jnp.asarray(np.repeat(np.eye(F, dtype=np.float32), D, axis=1))
    T = jnp.asarray(np.tile(np.eye(D, dtype=np.float32), (F, 1)))

    pred = _tc_mlp(
        emb, feat_vals, w, S, T,
        W0.astype(jnp.bfloat16), b0.reshape(1, -1),
        W1.astype(jnp.bfloat16), b1.reshape(1, -1),
        W2.astype(jnp.bfloat16), b2.reshape(1, -1), Wo, bfin.reshape(1, 1),
    )
    return pred.reshape(-1)
